# Initial kernel scaffold; baseline (speedup 1.0000x reference)
import jax
import jax.numpy as jnp
from jax.experimental import pallas as pl


def kernel(x, label, idx, embedding_weight):
    raise NotImplementedError("write your pallas kernel here")


# speedup vs baseline: 1.3652x; 1.3652x over previous
import functools

import jax
import jax.numpy as jnp
from jax import lax
from jax.experimental import pallas as pl
from jax.experimental.pallas import tpu as pltpu
from jax.experimental.pallas import tpu_sc as plsc

_N_TOK = 16384
_N_CODE = 8192
_DIM = 32
_BLK = 256
_GRID = _N_TOK // _BLK


def _dist_argmin_body(x_ref, embt_ref, idx_ref, losssum_ref, e2_ref):
    i = pl.program_id(0)

    @pl.when(i == 0)
    def _init():
        e2_ref[...] = jnp.sum(embt_ref[...] * embt_ref[...], axis=0,
                              keepdims=True)
        losssum_ref[0, 0] = 0.0

    x = x_ref[...]
    x2 = jnp.sum(x * x, axis=1, keepdims=True)
    xe = jnp.dot(x, embt_ref[...],
                 preferred_element_type=jnp.float32)
    d = (x2 + e2_ref[...]) - 2.0 * xe
    h = _N_CODE // 2
    d0, d1 = d[:, :h], d[:, h:]
    m0 = jnp.min(d0, axis=1, keepdims=True)
    m1 = jnp.min(d1, axis=1, keepdims=True)
    ii = lax.broadcasted_iota(jnp.int32, (_BLK, h), 1)
    i0 = jnp.min(jnp.where(d0 == m0, ii, _N_CODE), axis=1)
    i1 = jnp.min(jnp.where(d1 == m1, ii + h, _N_CODE), axis=1)
    b0 = m0.astype(jnp.bfloat16).astype(jnp.float32)
    win1 = (m1 < b0)[:, 0]
    idx_ref[...] = jnp.where(win1, i1, i0)
    chosen = jnp.where(win1, m1[:, 0], m0[:, 0])
    losssum_ref[0, 0] += jnp.sum(chosen)


def _dist_argmin(x, emb_t):
    return pl.pallas_call(
        _dist_argmin_body,
        grid=(_GRID,),
        in_specs=[
            pl.BlockSpec((_BLK, _DIM), lambda i: (i, 0)),
            pl.BlockSpec((_DIM, _N_CODE), lambda i: (0, 0)),
        ],
        out_specs=[
            pl.BlockSpec((_BLK,), lambda i: (i,)),
            pl.BlockSpec(memory_space=pltpu.SMEM),
        ],
        out_shape=[
            jax.ShapeDtypeStruct((_N_TOK,), jnp.int32),
            jax.ShapeDtypeStruct((1, 1), jnp.float32),
        ],
        scratch_shapes=[pltpu.VMEM((1, _N_CODE), jnp.float32)],
    )(x, emb_t)


@functools.cache
def _make_sc_gather():
    info = plsc.get_sparse_core_info()
    nc = info.num_cores
    nw = nc * info.num_subcores
    bpw = _N_TOK // nw

    @functools.partial(
        pl.kernel,
        mesh=plsc.VectorSubcoreMesh(core_axis_name="c", subcore_axis_name="s"),
        compiler_params=pltpu.CompilerParams(use_tc_tiling_on_sc=False),
        out_type=jax.ShapeDtypeStruct((_N_TOK, _DIM), jnp.float32),
        scratch_types=[
            pltpu.VMEM((bpw,), jnp.int32),
            pltpu.VMEM((bpw, _DIM), jnp.float32),
            pltpu.SemaphoreType.DMA,
        ],
    )
    def _sc_gather(table_hbm, idx_hbm, out_hbm, idx_v, rows_v, sem):
        wid = lax.axis_index("s") * nc + lax.axis_index("c")
        base = wid * bpw
        pltpu.sync_copy(idx_hbm.at[pl.ds(base, bpw)], idx_v)
        pltpu.async_copy(table_hbm.at[idx_v], rows_v, sem).wait()
        pltpu.sync_copy(rows_v, out_hbm.at[pl.ds(base, bpw)])

    return _sc_gather


def kernel(x, label, idx, embedding_weight):
    del label, idx
    emb_t = embedding_weight.T
    indices, losssum = _dist_argmin(x, emb_t)
    x_q = _make_sc_gather()(embedding_weight, indices)
    loss = losssum[0, 0] * (2.0 / (_N_TOK * _DIM))
    return (x_q, loss, indices)
